# baseline (speedup 1.0000x reference)
import jax
import jax.numpy as jnp
from jax import lax
from jax.experimental import pallas as pl
from jax.experimental.pallas import tpu as pltpu
from jax.experimental.pallas import tpu_sc as plsc

VOCAB = 1000
D_MODEL = 64
SEQ = 50
LANES = 16
BLK = 128
NUM_CORES = 2
NUM_SUBCORES = 16
NUM_WORKERS = NUM_CORES * NUM_SUBCORES
NBLK = 1024 // BLK
NQ = NUM_WORKERS // NBLK
TMAX = 14


def _transpose_block(slot, staging, iota16):
    rows = [iota16 + (c * LANES) for c in range(D_MODEL // LANES)]
    for b in range(BLK):
        col = jnp.full((LANES,), b, jnp.int32)
        for c in range(D_MODEL // LANES):
            val = slot[b, pl.ds(c * LANES, LANES)]
            plsc.store_scatter(staging, [rows[c], col], val)


def _emb_body(idx_hbm, table_hbm, out_hbm, idx_v, slot_a, slot_b,
              stg_a, stg_b, gsem, osem):
    wid = lax.axis_index("s") * NUM_CORES + lax.axis_index("c")
    m = wid % NBLK
    q = wid // NBLK
    s_start = q * 13 - jnp.maximum(0, q - 2)
    count = jnp.where(q >= 2, 12, 13)
    iota16 = lax.iota(jnp.int32, LANES)

    pltpu.sync_copy(idx_hbm.at[:, m], idx_v)

    def gather(t, slot):
        return pltpu.async_copy(table_hbm.at[idx_v.at[s_start + t]], slot, gsem)

    def drain_gather():
        pltpu.make_async_copy(table_hbm.at[pl.ds(0, BLK)], slot_a, gsem).wait()

    def put(t, stg):
        off = pl.multiple_of(m * BLK, BLK)
        return pltpu.async_copy(
            stg, out_hbm.at[s_start + t, :, pl.ds(off, BLK)], osem
        )

    def drain_put():
        pltpu.make_async_copy(
            table_hbm.at[pl.ds(0, D_MODEL)], stg_a, osem
        ).wait()

    gather(0, slot_a)
    gather(1, slot_b)

    def body(k, carry):
        for t, slot, stg in ((2 * k, slot_a, stg_a), (2 * k + 1, slot_b, stg_b)):
            valid = t < count

            @pl.when(valid)
            def _drain(t=t):
                drain_gather()

            @pl.when(valid & (t >= 2))
            def _free_stage():
                drain_put()

            @pl.when(valid)
            def _work(t=t, slot=slot, stg=stg):
                _transpose_block(slot, stg, iota16)
                put(t, stg)

            nt = t + 2

            @pl.when(nt < count)
            def _prefetch(nt=nt, slot=slot):
                gather(nt, slot)
        return carry

    lax.fori_loop(0, TMAX // 2, body, 0)
    drain_put()
    drain_put()


def kernel(token_ids, w):
    nxt = jnp.concatenate([w[1:], jnp.zeros((1, D_MODEL), w.dtype)], axis=0)
    table2 = jnp.concatenate([w, nxt], axis=1)
    ids_sdb = token_ids.T.reshape(SEQ, NBLK, BLK)
    grab = pl.kernel(
        _emb_body,
        out_type=jax.ShapeDtypeStruct((SEQ, D_MODEL, 1024), jnp.float32),
        mesh=plsc.VectorSubcoreMesh(
            core_axis_name="c",
            subcore_axis_name="s",
            num_cores=NUM_CORES,
            num_subcores=NUM_SUBCORES,
        ),
        scratch_types=[
            pltpu.VMEM((SEQ, BLK), jnp.int32),
            pltpu.VMEM((BLK, 2 * D_MODEL), jnp.float32),
            pltpu.VMEM((BLK, 2 * D_MODEL), jnp.float32),
            pltpu.VMEM((D_MODEL, BLK), jnp.float32),
            pltpu.VMEM((D_MODEL, BLK), jnp.float32),
            pltpu.SemaphoreType.DMA,
            pltpu.SemaphoreType.DMA,
        ],
        compiler_params=pltpu.CompilerParams(
            use_tc_tiling_on_sc=True,
            needs_layout_passes=False,
        ),
    )
    out = grab(ids_sdb, table2)
    return jnp.transpose(out, (2, 0, 1))


# speedup vs baseline: 1.3229x; 1.3229x over previous
import jax
import jax.numpy as jnp
from jax import lax
from jax.experimental import pallas as pl
from jax.experimental.pallas import tpu as pltpu
from jax.experimental.pallas import tpu_sc as plsc

VOCAB = 1000
D_MODEL = 64
SEQ = 50
LANES = 16
BLK = 128
NUM_CORES = 2
NUM_SUBCORES = 16
NUM_WORKERS = NUM_CORES * NUM_SUBCORES
NBLK = 1024 // BLK
NQ = NUM_WORKERS // NBLK
TMAX = 14


def _transpose_block(slot, staging, iota16):
    rows = [iota16 + (c * LANES) for c in range(D_MODEL // LANES)]

    @plsc.parallel_loop(0, BLK, unroll=8)
    def _body(b):
        col = jnp.broadcast_to(b, (LANES,)).astype(jnp.int32)
        for c in range(D_MODEL // LANES):
            val = slot[b, pl.ds(c * LANES, LANES)]
            plsc.store_scatter(staging, [rows[c], col], val)


def _emb_body(idx_hbm, table_hbm, out_hbm, idx_v, slot_a, slot_b,
              stg_a, stg_b, gsem, osem):
    wid = lax.axis_index("s") * NUM_CORES + lax.axis_index("c")
    m = wid % NBLK
    q = wid // NBLK
    s_start = q * 13 - jnp.maximum(0, q - 2)
    count = jnp.where(q >= 2, 12, 13)
    iota16 = lax.iota(jnp.int32, LANES)

    pltpu.sync_copy(idx_hbm.at[:, m], idx_v)

    def gather(t, slot):
        return pltpu.async_copy(table_hbm.at[idx_v.at[s_start + t]], slot, gsem)

    def drain_gather():
        pltpu.make_async_copy(table_hbm.at[pl.ds(0, BLK)], slot_a, gsem).wait()

    def put(t, stg):
        off = pl.multiple_of(m * BLK, BLK)
        return pltpu.async_copy(
            stg, out_hbm.at[s_start + t, :, pl.ds(off, BLK)], osem
        )

    def drain_put():
        pltpu.make_async_copy(
            table_hbm.at[pl.ds(0, D_MODEL)], stg_a, osem
        ).wait()

    gather(0, slot_a)
    gather(1, slot_b)

    def body(k, carry):
        for t, slot, stg in ((2 * k, slot_a, stg_a), (2 * k + 1, slot_b, stg_b)):
            valid = t < count

            @pl.when(valid)
            def _drain(t=t):
                drain_gather()

            @pl.when(valid & (t >= 2))
            def _free_stage():
                drain_put()

            @pl.when(valid)
            def _work(t=t, slot=slot, stg=stg):
                _transpose_block(slot, stg, iota16)
                put(t, stg)

            nt = t + 2

            @pl.when(nt < count)
            def _prefetch(nt=nt, slot=slot):
                gather(nt, slot)
        return carry

    lax.fori_loop(0, TMAX // 2, body, 0)
    drain_put()
    drain_put()


def kernel(token_ids, w):
    nxt = jnp.concatenate([w[1:], jnp.zeros((1, D_MODEL), w.dtype)], axis=0)
    table2 = jnp.concatenate([w, nxt], axis=1)
    ids_sdb = token_ids.T.reshape(SEQ, NBLK, BLK)
    grab = pl.kernel(
        _emb_body,
        out_type=jax.ShapeDtypeStruct((SEQ, D_MODEL, 1024), jnp.float32),
        mesh=plsc.VectorSubcoreMesh(
            core_axis_name="c",
            subcore_axis_name="s",
            num_cores=NUM_CORES,
            num_subcores=NUM_SUBCORES,
        ),
        scratch_types=[
            pltpu.VMEM((SEQ, BLK), jnp.int32),
            pltpu.VMEM((BLK, 2 * D_MODEL), jnp.float32),
            pltpu.VMEM((BLK, 2 * D_MODEL), jnp.float32),
            pltpu.VMEM((D_MODEL, BLK), jnp.float32),
            pltpu.VMEM((D_MODEL, BLK), jnp.float32),
            pltpu.SemaphoreType.DMA,
            pltpu.SemaphoreType.DMA,
        ],
        compiler_params=pltpu.CompilerParams(
            use_tc_tiling_on_sc=True,
            needs_layout_passes=False,
        ),
    )
    out = grab(ids_sdb, table2)
    return jnp.transpose(out, (2, 0, 1))


# speedup vs baseline: 1.3294x; 1.0049x over previous
import jax
import jax.numpy as jnp
from jax import lax
from jax.experimental import pallas as pl
from jax.experimental.pallas import tpu as pltpu
from jax.experimental.pallas import tpu_sc as plsc

VOCAB = 1000
D_MODEL = 64
SEQ = 50
LANES = 16
BLK = 128
NUM_CORES = 2
NUM_SUBCORES = 16
NUM_WORKERS = NUM_CORES * NUM_SUBCORES
NBLK = 1024 // BLK
NQ = NUM_WORKERS // NBLK
TMAX = 14


def _transpose_block(slot, staging, iota16):
    rows = [iota16 + (c * LANES) for c in range(D_MODEL // LANES)]

    @plsc.parallel_loop(0, BLK, unroll=8)
    def _body(b):
        col = jnp.broadcast_to(b, (LANES,)).astype(jnp.int32)
        for c in range(D_MODEL // LANES):
            val = slot[b, pl.ds(c * LANES, LANES)]
            plsc.store_scatter(staging, [rows[c], col], val)


def _emb_body(idx_hbm, table_hbm, out_hbm, idx_v, shared_tab, slot_a, slot_b,
              stg_a, stg_b, gsem, osem):
    sid = lax.axis_index("s")
    wid = sid * NUM_CORES + lax.axis_index("c")
    m = wid % NBLK
    q = wid // NBLK
    s_start = q * 13 - jnp.maximum(0, q - 2)
    count = jnp.where(q >= 2, 12, 13)
    iota16 = lax.iota(jnp.int32, LANES)

    @pl.when(sid == 0)
    def _stage_table():
        pltpu.sync_copy(table_hbm, shared_tab)

    pltpu.sync_copy(idx_hbm.at[:, m], idx_v)
    plsc.subcore_barrier()

    def gather(t, slot):
        return pltpu.async_copy(shared_tab.at[idx_v.at[s_start + t]], slot, gsem)

    def drain_gather():
        pltpu.make_async_copy(table_hbm.at[pl.ds(0, BLK)], slot_a, gsem).wait()

    def put(t, stg):
        off = pl.multiple_of(m * BLK, BLK)
        return pltpu.async_copy(
            stg, out_hbm.at[s_start + t, :, pl.ds(off, BLK)], osem
        )

    def drain_put():
        pltpu.make_async_copy(
            table_hbm.at[pl.ds(0, D_MODEL)], stg_a, osem
        ).wait()

    gather(0, slot_a)
    gather(1, slot_b)

    def body(k, carry):
        for t, slot, stg in ((2 * k, slot_a, stg_a), (2 * k + 1, slot_b, stg_b)):
            valid = t < count

            @pl.when(valid)
            def _drain(t=t):
                drain_gather()

            @pl.when(valid & (t >= 2))
            def _free_stage():
                drain_put()

            @pl.when(valid)
            def _work(t=t, slot=slot, stg=stg):
                _transpose_block(slot, stg, iota16)
                put(t, stg)

            nt = t + 2

            @pl.when(nt < count)
            def _prefetch(nt=nt, slot=slot):
                gather(nt, slot)
        return carry

    lax.fori_loop(0, TMAX // 2, body, 0)
    drain_put()
    drain_put()


def kernel(token_ids, w):
    nxt = jnp.concatenate([w[1:], jnp.zeros((1, D_MODEL), w.dtype)], axis=0)
    table2 = jnp.concatenate([w, nxt], axis=1)
    ids_sdb = token_ids.T.reshape(SEQ, NBLK, BLK)
    grab = pl.kernel(
        _emb_body,
        out_type=jax.ShapeDtypeStruct((SEQ, D_MODEL, 1024), jnp.float32),
        mesh=plsc.VectorSubcoreMesh(
            core_axis_name="c",
            subcore_axis_name="s",
            num_cores=NUM_CORES,
            num_subcores=NUM_SUBCORES,
        ),
        scratch_types=[
            pltpu.VMEM((SEQ, BLK), jnp.int32),
            pltpu.VMEM_SHARED((VOCAB, 2 * D_MODEL), jnp.float32),
            pltpu.VMEM((BLK, 2 * D_MODEL), jnp.float32),
            pltpu.VMEM((BLK, 2 * D_MODEL), jnp.float32),
            pltpu.VMEM((D_MODEL, BLK), jnp.float32),
            pltpu.VMEM((D_MODEL, BLK), jnp.float32),
            pltpu.SemaphoreType.DMA,
            pltpu.SemaphoreType.DMA,
        ],
        compiler_params=pltpu.CompilerParams(
            use_tc_tiling_on_sc=True,
            needs_layout_passes=False,
        ),
    )
    out = grab(ids_sdb, table2)
    return jnp.transpose(out, (2, 0, 1))


# speedup vs baseline: 2.4929x; 1.8752x over previous
import jax
import jax.numpy as jnp
from jax import lax
from jax.experimental import pallas as pl
from jax.experimental.pallas import tpu as pltpu
from jax.experimental.pallas import tpu_sc as plsc

VOCAB = 1000
D_MODEL = 64
SEQ = 50
LANES = 16
BLK = 128
NUM_CORES = 2
NUM_SUBCORES = 16
NUM_WORKERS = NUM_CORES * NUM_SUBCORES
NBLK = 1024 // BLK
NQ = NUM_WORKERS // NBLK
TMAX = 14


def _transpose_block(slot, staging, iota16):
    for bb in range(BLK // LANES):
        rowvec = iota16 + (bb * LANES)
        for dc in range(D_MODEL // LANES):

            @plsc.parallel_loop(0, LANES, unroll=8)
            def _body(k, rowvec=rowvec, dc=dc):
                perm = (iota16 + k) & (LANES - 1)
                colvec = perm + (dc * LANES)
                val = plsc.load_gather(slot, [rowvec, colvec])
                plsc.store_scatter(staging, [colvec, rowvec], val)


def _emb_body(idx_hbm, table_hbm, out_hbm, idx_v, shared_tab, slot_a, slot_b,
              stg_a, stg_b, gsem, osem):
    sid = lax.axis_index("s")
    wid = sid * NUM_CORES + lax.axis_index("c")
    m = wid % NBLK
    q = wid // NBLK
    s_start = q * 13 - jnp.maximum(0, q - 2)
    count = jnp.where(q >= 2, 12, 13)
    iota16 = lax.iota(jnp.int32, LANES)

    @pl.when(sid == 0)
    def _stage_table():
        pltpu.sync_copy(table_hbm, shared_tab)

    pltpu.sync_copy(idx_hbm.at[:, m], idx_v)
    plsc.subcore_barrier()

    def gather(t, slot):
        return pltpu.async_copy(shared_tab.at[idx_v.at[s_start + t]], slot, gsem)

    def drain_gather():
        pltpu.make_async_copy(table_hbm.at[pl.ds(0, BLK)], slot_a, gsem).wait()

    def put(t, stg):
        off = pl.multiple_of(m * BLK, BLK)
        return pltpu.async_copy(
            stg, out_hbm.at[s_start + t, :, pl.ds(off, BLK)], osem
        )

    def drain_put():
        pltpu.make_async_copy(
            table_hbm.at[pl.ds(0, D_MODEL)], stg_a, osem
        ).wait()

    gather(0, slot_a)
    gather(1, slot_b)

    def body(k, carry):
        for t, slot, stg in ((2 * k, slot_a, stg_a), (2 * k + 1, slot_b, stg_b)):
            valid = t < count

            @pl.when(valid)
            def _drain(t=t):
                drain_gather()

            @pl.when(valid & (t >= 2))
            def _free_stage():
                drain_put()

            @pl.when(valid)
            def _work(t=t, slot=slot, stg=stg):
                _transpose_block(slot, stg, iota16)
                put(t, stg)

            nt = t + 2

            @pl.when(nt < count)
            def _prefetch(nt=nt, slot=slot):
                gather(nt, slot)
        return carry

    lax.fori_loop(0, TMAX // 2, body, 0)
    drain_put()
    drain_put()


def kernel(token_ids, w):
    nxt = jnp.concatenate([w[1:], jnp.zeros((1, D_MODEL), w.dtype)], axis=0)
    table2 = jnp.concatenate([w, nxt], axis=1)
    ids_sdb = token_ids.T.reshape(SEQ, NBLK, BLK)
    grab = pl.kernel(
        _emb_body,
        out_type=jax.ShapeDtypeStruct((SEQ, D_MODEL, 1024), jnp.float32),
        mesh=plsc.VectorSubcoreMesh(
            core_axis_name="c",
            subcore_axis_name="s",
            num_cores=NUM_CORES,
            num_subcores=NUM_SUBCORES,
        ),
        scratch_types=[
            pltpu.VMEM((SEQ, BLK), jnp.int32),
            pltpu.VMEM_SHARED((VOCAB, 2 * D_MODEL), jnp.float32),
            pltpu.VMEM((BLK, 2 * D_MODEL), jnp.float32),
            pltpu.VMEM((BLK, 2 * D_MODEL), jnp.float32),
            pltpu.VMEM((D_MODEL, BLK), jnp.float32),
            pltpu.VMEM((D_MODEL, BLK), jnp.float32),
            pltpu.SemaphoreType.DMA,
            pltpu.SemaphoreType.DMA,
        ],
        compiler_params=pltpu.CompilerParams(
            use_tc_tiling_on_sc=True,
            needs_layout_passes=False,
        ),
    )
    out = grab(ids_sdb, table2)
    return jnp.transpose(out, (2, 0, 1))


# speedup vs baseline: 2.5922x; 1.0399x over previous
import jax
import jax.numpy as jnp
from jax import lax
from jax.experimental import pallas as pl
from jax.experimental.pallas import tpu as pltpu
from jax.experimental.pallas import tpu_sc as plsc

VOCAB = 1000
D_MODEL = 64
SEQ = 50
LANES = 16
BLK = 128
NUM_CORES = 2
NUM_SUBCORES = 16
NUM_WORKERS = NUM_CORES * NUM_SUBCORES
NBLK = 1024 // BLK
NQ = NUM_WORKERS // NBLK
TMAX = 14


def _transpose_block(slot, staging, iota16):
    rowvecs = [iota16 + (bb * LANES) for bb in range(BLK // LANES)]

    @plsc.parallel_loop(0, LANES, unroll=4)
    def _body(k):
        perm = (iota16 + k) & (LANES - 1)
        for dc in range(D_MODEL // LANES):
            colvec = perm + (dc * LANES)
            for rowvec in rowvecs:
                val = plsc.load_gather(slot, [rowvec, colvec])
                plsc.store_scatter(staging, [colvec, rowvec], val)


def _emb_body(idx_hbm, table_hbm, out_hbm, idx_v, shared_tab, slot_a, slot_b,
              stg_a, stg_b, gsem, osem):
    sid = lax.axis_index("s")
    wid = sid * NUM_CORES + lax.axis_index("c")
    m = wid % NBLK
    q = wid // NBLK
    s_start = q * 13 - jnp.maximum(0, q - 2)
    count = jnp.where(q >= 2, 12, 13)
    iota16 = lax.iota(jnp.int32, LANES)

    @pl.when(sid == 0)
    def _stage_table():
        pltpu.sync_copy(table_hbm, shared_tab)

    pltpu.sync_copy(idx_hbm.at[:, m], idx_v)
    plsc.subcore_barrier()

    def gather(t, slot):
        return pltpu.async_copy(shared_tab.at[idx_v.at[s_start + t]], slot, gsem)

    def drain_gather():
        pltpu.make_async_copy(table_hbm.at[pl.ds(0, BLK)], slot_a, gsem).wait()

    def put(t, stg):
        off = pl.multiple_of(m * BLK, BLK)
        return pltpu.async_copy(
            stg, out_hbm.at[s_start + t, :, pl.ds(off, BLK)], osem
        )

    def drain_put():
        pltpu.make_async_copy(
            table_hbm.at[pl.ds(0, D_MODEL)], stg_a, osem
        ).wait()

    gather(0, slot_a)
    gather(1, slot_b)

    def body(k, carry):
        for t, slot, stg in ((2 * k, slot_a, stg_a), (2 * k + 1, slot_b, stg_b)):
            valid = t < count

            @pl.when(valid)
            def _drain(t=t):
                drain_gather()

            @pl.when(valid & (t >= 2))
            def _free_stage():
                drain_put()

            @pl.when(valid)
            def _work(t=t, slot=slot, stg=stg):
                _transpose_block(slot, stg, iota16)
                put(t, stg)

            nt = t + 2

            @pl.when(nt < count)
            def _prefetch(nt=nt, slot=slot):
                gather(nt, slot)
        return carry

    lax.fori_loop(0, TMAX // 2, body, 0)
    drain_put()
    drain_put()


def kernel(token_ids, w):
    nxt = jnp.concatenate([w[1:], jnp.zeros((1, D_MODEL), w.dtype)], axis=0)
    table2 = jnp.concatenate([w, nxt], axis=1)
    ids_sdb = token_ids.T.reshape(SEQ, NBLK, BLK)
    grab = pl.kernel(
        _emb_body,
        out_type=jax.ShapeDtypeStruct((SEQ, D_MODEL, 1024), jnp.float32),
        mesh=plsc.VectorSubcoreMesh(
            core_axis_name="c",
            subcore_axis_name="s",
            num_cores=NUM_CORES,
            num_subcores=NUM_SUBCORES,
        ),
        scratch_types=[
            pltpu.VMEM((SEQ, BLK), jnp.int32),
            pltpu.VMEM_SHARED((VOCAB, 2 * D_MODEL), jnp.float32),
            pltpu.VMEM((BLK, 2 * D_MODEL), jnp.float32),
            pltpu.VMEM((BLK, 2 * D_MODEL), jnp.float32),
            pltpu.VMEM((D_MODEL, BLK), jnp.float32),
            pltpu.VMEM((D_MODEL, BLK), jnp.float32),
            pltpu.SemaphoreType.DMA,
            pltpu.SemaphoreType.DMA,
        ],
        compiler_params=pltpu.CompilerParams(
            use_tc_tiling_on_sc=True,
            needs_layout_passes=False,
        ),
    )
    out = grab(ids_sdb, table2)
    return jnp.transpose(out, (2, 0, 1))
